# baseline (speedup 1.0000x reference)
import functools

import jax
import jax.numpy as jnp
from jax import lax
from jax.experimental import pallas as pl
from jax.experimental.pallas import tpu as pltpu
from jax.experimental.pallas import tpu_sc as plsc

_NUM_NODES = 1000000
_EMBED_DIM = 64
_NUM_CORES = 2
_NUM_SUBCORES = 16
_NUM_WORKERS = _NUM_CORES * _NUM_SUBCORES
_ROWS_PER_W = (_NUM_NODES // _NUM_WORKERS) // 8 * 8
_TAIL_BASE = _ROWS_PER_W * _NUM_WORKERS
_TAIL_ROWS = _NUM_NODES - _TAIL_BASE

_NBUF = 3
_CHUNK = 248
_NCHUNKS = _ROWS_PER_W // _CHUNK
_NGROUPS = _NCHUNKS // _NBUF
assert _CHUNK * _NCHUNKS == _ROWS_PER_W and _NBUF * _NGROUPS == _NCHUNKS

_MESH = plsc.VectorSubcoreMesh(core_axis_name="c", subcore_axis_name="s")


@functools.partial(
    pl.kernel,
    out_type=jax.ShapeDtypeStruct((_NUM_NODES, _EMBED_DIM), jnp.float32),
    mesh=_MESH,
    compiler_params=pltpu.CompilerParams(use_tc_tiling_on_sc=False),
    scratch_types=[
        [pltpu.VMEM((_CHUNK, _EMBED_DIM), jnp.float32) for _ in range(_NBUF)],
        [pltpu.SemaphoreType.DMA for _ in range(_NBUF)],
        [pltpu.SemaphoreType.DMA for _ in range(_NBUF)],
    ],
)
def _sc_copy(w_hbm, o_hbm, bufs, in_sems, out_sems):
    wid = lax.axis_index("s") * _NUM_CORES + lax.axis_index("c")
    base = pl.multiple_of(wid * _ROWS_PER_W, 8)

    def _in_copy(k, b):
        off = pl.multiple_of(base + k * _CHUNK, 8)
        return pltpu.make_async_copy(
            w_hbm.at[pl.ds(off, _CHUNK)], bufs[b], in_sems[b])

    def _out_copy(k, b):
        off = pl.multiple_of(base + k * _CHUNK, 8)
        return pltpu.make_async_copy(
            bufs[b], o_hbm.at[pl.ds(off, _CHUNK)], out_sems[b])

    for j in range(_NBUF):
        _in_copy(j, j).start()

    def _group(g, carry):
        for j in range(_NBUF):
            k = g * _NBUF + j
            _in_copy(k, j).wait()
            _out_copy(k, j).start()
        for j in range(_NBUF):
            k = g * _NBUF + j

            @pl.when(k + _NBUF < _NCHUNKS)
            def _():
                _out_copy(k, j).wait()
                _in_copy(k + _NBUF, j).start()

        return carry

    lax.fori_loop(0, _NGROUPS, _group, 0)

    for j in range(_NBUF):
        _out_copy(_NCHUNKS - _NBUF + j, j).wait()

    @pl.when(wid == 0)
    def _():
        pltpu.make_async_copy(
            w_hbm.at[pl.ds(_TAIL_BASE, _TAIL_ROWS)],
            bufs[0].at[pl.ds(0, _TAIL_ROWS)], in_sems[0]).start()
        pltpu.make_async_copy(
            w_hbm.at[pl.ds(_TAIL_BASE, _TAIL_ROWS)],
            bufs[0].at[pl.ds(0, _TAIL_ROWS)], in_sems[0]).wait()
        pltpu.make_async_copy(
            bufs[0].at[pl.ds(0, _TAIL_ROWS)],
            o_hbm.at[pl.ds(_TAIL_BASE, _TAIL_ROWS)], out_sems[0]).start()
        pltpu.make_async_copy(
            bufs[0].at[pl.ds(0, _TAIL_ROWS)],
            o_hbm.at[pl.ds(_TAIL_BASE, _TAIL_ROWS)], out_sems[0]).wait()


def kernel(weight):
    return _sc_copy(weight)


# speedup vs baseline: 6.8391x; 6.8391x over previous
import functools

import jax
import jax.numpy as jnp
from jax import lax
from jax.experimental import pallas as pl
from jax.experimental.pallas import tpu as pltpu
from jax.experimental.pallas import tpu_sc as plsc

_NUM_NODES = 1000000
_EMBED_DIM = 64
_NUM_CORES = 2
_NUM_SUBCORES = 16
_NUM_WORKERS = _NUM_CORES * _NUM_SUBCORES

_CHUNK = 512
_NCHUNKS = 61
_COLS_PER_W = _CHUNK * _NCHUNKS
_TAIL_BASE = _COLS_PER_W * _NUM_WORKERS
_TAIL_COLS = _NUM_NODES - _TAIL_BASE
_NBUF = 3

_MESH = plsc.VectorSubcoreMesh(core_axis_name="c", subcore_axis_name="s")


@functools.partial(
    pl.kernel,
    out_type=jax.ShapeDtypeStruct((_EMBED_DIM, _NUM_NODES), jnp.float32),
    mesh=_MESH,
    scratch_types=[
        [pltpu.VMEM((_EMBED_DIM, _CHUNK), jnp.float32) for _ in range(_NBUF)],
        [pltpu.SemaphoreType.DMA for _ in range(_NBUF)],
        [pltpu.SemaphoreType.DMA for _ in range(_NBUF)],
    ],
)
def _sc_copy(w_hbm, o_hbm, bufs, in_sems, out_sems):
    wid = lax.axis_index("s") * _NUM_CORES + lax.axis_index("c")
    base = pl.multiple_of(wid * _COLS_PER_W, 128)

    def _in_copy(k, b):
        off = pl.multiple_of(base + k * _CHUNK, 128)
        return pltpu.make_async_copy(
            w_hbm.at[:, pl.ds(off, _CHUNK)], bufs[b], in_sems[b])

    def _out_copy(k, b):
        off = pl.multiple_of(base + k * _CHUNK, 128)
        return pltpu.make_async_copy(
            bufs[b], o_hbm.at[:, pl.ds(off, _CHUNK)], out_sems[b])

    _in_copy(0, 0).start()
    _in_copy(1, 1).start()
    for k in range(_NCHUNKS):
        b = k % _NBUF
        _in_copy(k, b).wait()
        _out_copy(k, b).start()
        if k + 2 < _NCHUNKS:
            if k >= 1:
                _out_copy(k - 1, (k + 2) % _NBUF).wait()
            _in_copy(k + 2, (k + 2) % _NBUF).start()
    for k in range(_NCHUNKS - _NBUF, _NCHUNKS):
        _out_copy(k, k % _NBUF).wait()

    @pl.when(wid == 0)
    def _():
        pltpu.make_async_copy(
            w_hbm.at[:, pl.ds(_TAIL_BASE, 512)], bufs[0], in_sems[0]).start()
        pltpu.make_async_copy(
            w_hbm.at[:, pl.ds(_TAIL_BASE, 512)], bufs[0], in_sems[0]).wait()
        pltpu.make_async_copy(
            bufs[0], o_hbm.at[:, pl.ds(_TAIL_BASE, 512)], out_sems[0]).start()
        pltpu.make_async_copy(
            bufs[0], o_hbm.at[:, pl.ds(_TAIL_BASE, 512)], out_sems[0]).wait()


def kernel(weight):
    out_t = _sc_copy(weight.T)
    tail = lax.slice(weight, (_TAIL_BASE + 512, 0), (_NUM_NODES, _EMBED_DIM))
    out_t = lax.dynamic_update_slice(out_t, tail.T, (0, _TAIL_BASE + 512))
    return out_t.T


# speedup vs baseline: 6.9514x; 1.0164x over previous
import functools

import jax
import jax.numpy as jnp
from jax import lax
from jax.experimental import pallas as pl
from jax.experimental.pallas import tpu as pltpu
from jax.experimental.pallas import tpu_sc as plsc

_NUM_NODES = 1000000
_EMBED_DIM = 64
_NUM_CORES = 2
_NUM_SUBCORES = 16
_NUM_WORKERS = _NUM_CORES * _NUM_SUBCORES

_NBANDS = 8
_NQ = 4
_COLS_PER_W = 249984
_TAIL_BASE = _COLS_PER_W * _NQ
_CHUNK = 3968
_NCHUNKS = _COLS_PER_W // _CHUNK
_NBUF = 4

_MESH = plsc.VectorSubcoreMesh(core_axis_name="c", subcore_axis_name="s")


@functools.partial(
    pl.kernel,
    out_type=jax.ShapeDtypeStruct((_EMBED_DIM, _NUM_NODES), jnp.float32),
    mesh=_MESH,
    scratch_types=[
        [pltpu.VMEM((8, _CHUNK), jnp.float32) for _ in range(_NBUF)],
        [pltpu.SemaphoreType.DMA for _ in range(_NBUF)],
        [pltpu.SemaphoreType.DMA for _ in range(_NBUF)],
    ],
)
def _sc_copy(w_hbm, o_hbm, bufs, in_sems, out_sems):
    wid = lax.axis_index("s") * _NUM_CORES + lax.axis_index("c")
    band = wid // _NQ
    row = pl.multiple_of(band * 8, 8)
    cbase = pl.multiple_of((wid % _NQ) * _COLS_PER_W, 128)

    def _in_copy(k, b):
        off = pl.multiple_of(cbase + k * _CHUNK, 128)
        return pltpu.make_async_copy(
            w_hbm.at[pl.ds(row, 8), pl.ds(off, _CHUNK)], bufs[b], in_sems[b])

    def _out_copy(k, b):
        off = pl.multiple_of(cbase + k * _CHUNK, 128)
        return pltpu.make_async_copy(
            bufs[b], o_hbm.at[pl.ds(row, 8), pl.ds(off, _CHUNK)], out_sems[b])

    for j in range(_NBUF - 1):
        _in_copy(j, j).start()
    for k in range(_NCHUNKS):
        b = k % _NBUF
        _in_copy(k, b).wait()
        _out_copy(k, b).start()
        if k + _NBUF - 1 < _NCHUNKS:
            if k >= 1:
                _out_copy(k - 1, (k + _NBUF - 1) % _NBUF).wait()
            _in_copy(k + _NBUF - 1, (k + _NBUF - 1) % _NBUF).start()
    for k in range(_NCHUNKS - _NBUF, _NCHUNKS):
        _out_copy(k, k % _NBUF).wait()


def kernel(weight):
    out_t = _sc_copy(weight.T)
    tail = lax.slice(weight, (_TAIL_BASE, 0), (_NUM_NODES, _EMBED_DIM))
    out_t = lax.dynamic_update_slice(out_t, tail.T, (0, _TAIL_BASE))
    return out_t.T
